# baseline (speedup 1.0000x reference)
import functools
import jax
import jax.numpy as jnp
from jax import lax
from jax.experimental import pallas as pl
from jax.experimental.pallas import tpu as pltpu

D = 2048
NH = 16
DH = D // NH
S = 2048
B = 2
NE = 3
NT = S * B
EPS = 1e-5
SCALE = 1.0 / (DH ** 0.5)
NEG = -1e30


def _ln1_body(x_ref, w_ref, b_ref, o_ref):
    x = x_ref[...]
    m = jnp.mean(x, axis=-1, keepdims=True)
    v = jnp.mean((x - m) ** 2, axis=-1, keepdims=True)
    y = (x - m) * lax.rsqrt(v + EPS) * w_ref[...] + b_ref[...]
    o_ref[...] = y.astype(jnp.bfloat16)


def _ln1(x32, w, b, bm=512):
    return pl.pallas_call(
        _ln1_body,
        grid=(NT // bm,),
        in_specs=[
            pl.BlockSpec((bm, D), lambda i: (i, 0)),
            pl.BlockSpec((1, D), lambda i: (0, 0)),
            pl.BlockSpec((1, D), lambda i: (0, 0)),
        ],
        out_specs=pl.BlockSpec((bm, D), lambda i: (i, 0)),
        out_shape=jax.ShapeDtypeStruct((NT, D), jnp.bfloat16),
    )(x32, w, b)


def _qkv_body(a_ref, w_ref, b_ref, o_ref):
    a = a_ref[...]
    w = w_ref[...].astype(jnp.bfloat16)
    acc = lax.dot_general(a, w, (((1,), (1,)), ((), ())),
                          preferred_element_type=jnp.float32)
    acc = acc + b_ref[...]
    o_ref[...] = acc.astype(jnp.bfloat16)


def _qkv(xn, in_w, in_b2, bm=512, bn=1024):
    return pl.pallas_call(
        _qkv_body,
        grid=(3 * D // bn, NT // bm),
        in_specs=[
            pl.BlockSpec((bm, D), lambda cb, rb: (rb, 0)),
            pl.BlockSpec((bn, D), lambda cb, rb: (cb, 0)),
            pl.BlockSpec((1, bn), lambda cb, rb: (0, cb)),
        ],
        out_specs=pl.BlockSpec((bm, bn), lambda cb, rb: (rb, cb)),
        out_shape=jax.ShapeDtypeStruct((NT, 3 * D), jnp.bfloat16),
    )(xn, in_w, in_b2)


def _attn_body(q_ref, k_ref, v_ref, o_ref, *, bq):
    qi = pl.program_id(2)
    q = q_ref[:, 0, 0, 0, :]
    nsteps = qi + 1

    def step(j, carry):
        o, m, l = carry
        kb = k_ref[pl.ds(j * bq, bq), 0, 0, 0, :]
        vb = v_ref[pl.ds(j * bq, bq), 0, 0, 0, :]
        s = lax.dot_general(q, kb, (((1,), (1,)), ((), ())),
                            preferred_element_type=jnp.float32) * SCALE
        row = qi * bq + lax.broadcasted_iota(jnp.int32, (bq, bq), 0)
        col = j * bq + lax.broadcasted_iota(jnp.int32, (bq, bq), 1)
        s = jnp.where(col <= row, s, NEG)
        m_new = jnp.maximum(m, jnp.max(s, axis=-1))
        alpha = jnp.exp(m - m_new)
        p = jnp.exp(s - m_new[:, None])
        l_new = l * alpha + jnp.sum(p, axis=-1)
        o_new = o * alpha[:, None] + lax.dot_general(
            p.astype(jnp.bfloat16), vb, (((1,), (0,)), ((), ())),
            preferred_element_type=jnp.float32)
        return o_new, m_new, l_new

    o0 = jnp.zeros((bq, DH), jnp.float32)
    m0 = jnp.full((bq,), NEG, jnp.float32)
    l0 = jnp.zeros((bq,), jnp.float32)
    o, m, l = lax.fori_loop(0, nsteps, step, (o0, m0, l0))
    o_ref[:, 0, 0, 0, :] = (o / l[:, None]).astype(jnp.bfloat16)


def _attn(qkv5, bq=512):
    return pl.pallas_call(
        functools.partial(_attn_body, bq=bq),
        grid=(B, NH, S // bq),
        in_specs=[
            pl.BlockSpec((bq, 1, 1, 1, DH), lambda b, h, qi: (qi, b, h, 0, 0)),
            pl.BlockSpec((S, 1, 1, 1, DH), lambda b, h, qi: (0, b, NH + h, 0, 0)),
            pl.BlockSpec((S, 1, 1, 1, DH), lambda b, h, qi: (0, b, 2 * NH + h, 0, 0)),
        ],
        out_specs=pl.BlockSpec((bq, 1, 1, 1, DH), lambda b, h, qi: (qi, b, h, 0, 0)),
        out_shape=jax.ShapeDtypeStruct((S, B, NH, 1, DH), jnp.bfloat16),
    )(qkv5, qkv5, qkv5)


def _final_body(o_ref, w_ref, b_ref, x_ref, lw_ref, lb_ref, g_ref,
                out_ref, logit_ref):
    a = o_ref[...]
    w = w_ref[...].astype(jnp.bfloat16)
    h = lax.dot_general(a, w, (((1,), (1,)), ((), ())),
                        preferred_element_type=jnp.float32)
    h = h + b_ref[...] + x_ref[...]
    m = jnp.mean(h, axis=-1, keepdims=True)
    v = jnp.mean((h - m) ** 2, axis=-1, keepdims=True)
    hs = (h - m) * lax.rsqrt(v + EPS) * lw_ref[...] + lb_ref[...]
    gp = g_ref[...]
    logits = lax.dot_general(hs, gp, (((1,), (1,)), ((), ())),
                             preferred_element_type=jnp.float32)
    logit_ref[...] = logits
    l0 = logits[:, 0:1]
    l1 = logits[:, 1:2]
    l2 = logits[:, 2:3]
    top = jnp.maximum(jnp.maximum(l0, l1), l2)
    bot = jnp.minimum(jnp.minimum(l0, l1), l2)
    mid = l0 + l1 + l2 - top - bot
    gate = 1.0 / (1.0 + jnp.exp(mid - top))
    out_ref[...] = h * gate


def _final(o2, out_w, out_b2, x32, ln2_w2, ln2_b2, gate_pad, bm=256):
    return pl.pallas_call(
        _final_body,
        grid=(NT // bm,),
        in_specs=[
            pl.BlockSpec((bm, D), lambda i: (i, 0)),
            pl.BlockSpec((D, D), lambda i: (0, 0)),
            pl.BlockSpec((1, D), lambda i: (0, 0)),
            pl.BlockSpec((bm, D), lambda i: (i, 0)),
            pl.BlockSpec((1, D), lambda i: (0, 0)),
            pl.BlockSpec((1, D), lambda i: (0, 0)),
            pl.BlockSpec((128, D), lambda i: (0, 0)),
        ],
        out_specs=[
            pl.BlockSpec((bm, D), lambda i: (i, 0)),
            pl.BlockSpec((bm, 128), lambda i: (i, 0)),
        ],
        out_shape=[
            jax.ShapeDtypeStruct((NT, D), jnp.float32),
            jax.ShapeDtypeStruct((NT, 128), jnp.float32),
        ],
    )(o2, out_w, out_b2, x32, ln2_w2, ln2_b2, gate_pad)


def kernel(x, ln1_w, ln1_b, in_w, in_b, out_w, out_b, ln2_w, ln2_b, gate_w):
    x32 = x.astype(jnp.float32)
    xf = x32.reshape(NT, D)
    xn = _ln1(xf, ln1_w.reshape(1, D), ln1_b.reshape(1, D))
    qkv = _qkv(xn, in_w, in_b.reshape(1, 3 * D))
    qkv5 = qkv.reshape(S, B, 3 * NH, 1, DH)
    o = _attn(qkv5)
    o2 = o.reshape(NT, D)
    gate_pad = jnp.zeros((128, D), jnp.float32).at[:NE].set(gate_w)
    out, logits_pad = _final(o2, out_w, out_b.reshape(1, D), xf,
                             ln2_w.reshape(1, D), ln2_b.reshape(1, D), gate_pad)
    return out.reshape(S, B, D), logits_pad[:, :NE]


# speedup vs baseline: 1.0148x; 1.0148x over previous
import functools
import jax
import jax.numpy as jnp
from jax import lax
from jax.experimental import pallas as pl
from jax.experimental.pallas import tpu as pltpu

D = 2048
NH = 16
DH = D // NH
S = 2048
B = 2
NE = 3
NT = S * B
EPS = 1e-5
SCALE = 1.0 / (DH ** 0.5)
NEG = -1e30


def _ln1_body(x_ref, w_ref, b_ref, o_ref):
    x = x_ref[...]
    m = jnp.mean(x, axis=-1, keepdims=True)
    v = jnp.mean((x - m) ** 2, axis=-1, keepdims=True)
    y = (x - m) * lax.rsqrt(v + EPS) * w_ref[...] + b_ref[...]
    o_ref[...] = y.astype(jnp.bfloat16)


def _ln1(x32, w, b, bm=512):
    return pl.pallas_call(
        _ln1_body,
        grid=(NT // bm,),
        in_specs=[
            pl.BlockSpec((bm, D), lambda i: (i, 0)),
            pl.BlockSpec((1, D), lambda i: (0, 0)),
            pl.BlockSpec((1, D), lambda i: (0, 0)),
        ],
        out_specs=pl.BlockSpec((bm, D), lambda i: (i, 0)),
        out_shape=jax.ShapeDtypeStruct((NT, D), jnp.bfloat16),
    )(x32, w, b)


def _qkv_body(a_ref, w_ref, b_ref, o_ref):
    a = a_ref[...]
    w = w_ref[...]
    acc = lax.dot_general(a, w, (((1,), (1,)), ((), ())),
                          preferred_element_type=jnp.float32)
    acc = acc + b_ref[...]
    o_ref[...] = acc.astype(jnp.bfloat16)


def _qkv(xn, in_w, in_b2, bm=512, bn=1024):
    return pl.pallas_call(
        _qkv_body,
        grid=(3 * D // bn, NT // bm),
        in_specs=[
            pl.BlockSpec((bm, D), lambda cb, rb: (rb, 0)),
            pl.BlockSpec((bn, D), lambda cb, rb: (cb, 0)),
            pl.BlockSpec((1, bn), lambda cb, rb: (0, cb)),
        ],
        out_specs=pl.BlockSpec((bm, bn), lambda cb, rb: (rb, cb)),
        out_shape=jax.ShapeDtypeStruct((NT, 3 * D), jnp.bfloat16),
    )(xn, in_w, in_b2)


def _attn_body(q_ref, k_ref, v_ref, o_ref, *, bq):
    qi = pl.program_id(2)
    q = q_ref[:, 0, 0, 0, :]

    def block(j, carry, masked):
        o, m, l = carry
        kb = k_ref[pl.ds(j * bq, bq), 0, 0, 0, :]
        vb = v_ref[pl.ds(j * bq, bq), 0, 0, 0, :]
        s = lax.dot_general(q, kb, (((1,), (1,)), ((), ())),
                            preferred_element_type=jnp.float32) * SCALE
        if masked:
            row = lax.broadcasted_iota(jnp.int32, (bq, bq), 0)
            col = lax.broadcasted_iota(jnp.int32, (bq, bq), 1)
            s = jnp.where(col <= row, s, NEG)
        m_new = jnp.maximum(m, jnp.max(s, axis=-1, keepdims=True))
        alpha = jnp.exp(m - m_new)
        p = jnp.exp(s - m_new)
        l_new = l * alpha + jnp.sum(p, axis=-1, keepdims=True)
        o_new = o * alpha + lax.dot_general(
            p.astype(jnp.bfloat16), vb, (((1,), (0,)), ((), ())),
            preferred_element_type=jnp.float32)
        return o_new, m_new, l_new

    o0 = jnp.zeros((bq, DH), jnp.float32)
    m0 = jnp.full((bq, 1), NEG, jnp.float32)
    l0 = jnp.zeros((bq, 1), jnp.float32)
    carry = lax.fori_loop(0, qi, lambda j, c: block(j, c, False), (o0, m0, l0))
    o, m, l = block(qi, carry, True)
    o_ref[:, 0, 0, 0, :] = (o * (1.0 / l)).astype(jnp.bfloat16)


def _attn(qkv5, bq=512):
    return pl.pallas_call(
        functools.partial(_attn_body, bq=bq),
        grid=(B, NH, S // bq),
        in_specs=[
            pl.BlockSpec((bq, 1, 1, 1, DH), lambda b, h, qi: (qi, b, h, 0, 0)),
            pl.BlockSpec((S, 1, 1, 1, DH), lambda b, h, qi: (0, b, NH + h, 0, 0)),
            pl.BlockSpec((S, 1, 1, 1, DH), lambda b, h, qi: (0, b, 2 * NH + h, 0, 0)),
        ],
        out_specs=pl.BlockSpec((bq, 1, 1, 1, DH), lambda b, h, qi: (qi, b, h, 0, 0)),
        out_shape=jax.ShapeDtypeStruct((S, B, NH, 1, DH), jnp.bfloat16),
    )(qkv5, qkv5, qkv5)


def _final_body(o_ref, w_ref, b_ref, x_ref, lw_ref, lb_ref, g_ref,
                out_ref, logit_ref):
    a = o_ref[...]
    w = w_ref[...]
    h = lax.dot_general(a, w, (((1,), (1,)), ((), ())),
                        preferred_element_type=jnp.float32)
    h = h + b_ref[...] + x_ref[...]
    m = jnp.mean(h, axis=-1, keepdims=True)
    v = jnp.mean((h - m) ** 2, axis=-1, keepdims=True)
    hs = (h - m) * lax.rsqrt(v + EPS) * lw_ref[...] + lb_ref[...]
    gp = g_ref[...]
    logits = lax.dot_general(hs, gp, (((1,), (1,)), ((), ())),
                             preferred_element_type=jnp.float32)
    logit_ref[...] = logits
    l0 = logits[:, 0:1]
    l1 = logits[:, 1:2]
    l2 = logits[:, 2:3]
    top = jnp.maximum(jnp.maximum(l0, l1), l2)
    bot = jnp.minimum(jnp.minimum(l0, l1), l2)
    mid = l0 + l1 + l2 - top - bot
    gate = 1.0 / (1.0 + jnp.exp(mid - top))
    out_ref[...] = h * gate


def _final(o2, out_w, out_b2, x32, ln2_w2, ln2_b2, gate_pad, bm=256):
    return pl.pallas_call(
        _final_body,
        grid=(NT // bm,),
        in_specs=[
            pl.BlockSpec((bm, D), lambda i: (i, 0)),
            pl.BlockSpec((D, D), lambda i: (0, 0)),
            pl.BlockSpec((1, D), lambda i: (0, 0)),
            pl.BlockSpec((bm, D), lambda i: (i, 0)),
            pl.BlockSpec((1, D), lambda i: (0, 0)),
            pl.BlockSpec((1, D), lambda i: (0, 0)),
            pl.BlockSpec((128, D), lambda i: (0, 0)),
        ],
        out_specs=[
            pl.BlockSpec((bm, D), lambda i: (i, 0)),
            pl.BlockSpec((bm, 128), lambda i: (i, 0)),
        ],
        out_shape=[
            jax.ShapeDtypeStruct((NT, D), jnp.float32),
            jax.ShapeDtypeStruct((NT, 128), jnp.float32),
        ],
    )(o2, out_w, out_b2, x32, ln2_w2, ln2_b2, gate_pad)


def kernel(x, ln1_w, ln1_b, in_w, in_b, out_w, out_b, ln2_w, ln2_b, gate_w):
    x32 = x.astype(jnp.float32)
    xf = x32.reshape(NT, D)
    xn = _ln1(xf, ln1_w.reshape(1, D), ln1_b.reshape(1, D))
    qkv = _qkv(xn, in_w.astype(jnp.bfloat16), in_b.reshape(1, 3 * D))
    qkv5 = qkv.reshape(S, B, 3 * NH, 1, DH)
    o = _attn(qkv5)
    o2 = o.reshape(NT, D)
    gate_pad = jnp.zeros((128, D), jnp.float32).at[:NE].set(gate_w)
    out, logits_pad = _final(o2, out_w.astype(jnp.bfloat16), out_b.reshape(1, D), xf,
                             ln2_w.reshape(1, D), ln2_b.reshape(1, D), gate_pad)
    return out.reshape(S, B, D), logits_pad[:, :NE]


# speedup vs baseline: 1.6997x; 1.6748x over previous
import functools
import jax
import jax.numpy as jnp
from jax import lax
from jax.experimental import pallas as pl
from jax.experimental.pallas import tpu as pltpu

D = 2048
NH = 16
DH = D // NH
S = 2048
B = 2
NE = 3
NT = S * B
EPS = 1e-5
SCALE = 1.0 / (DH ** 0.5)
NEG = -1e30
BQ = 512
NKB = S // BQ


def _ln1_body(x_ref, w_ref, b_ref, o_ref):
    x = x_ref[...]
    m = jnp.mean(x, axis=-1, keepdims=True)
    v = jnp.mean((x - m) ** 2, axis=-1, keepdims=True)
    y = (x - m) * lax.rsqrt(v + EPS) * w_ref[...] + b_ref[...]
    o_ref[...] = y.astype(jnp.bfloat16)


def _ln1(x2d, w, b, bm=512):
    nsb = S // bm
    return pl.pallas_call(
        _ln1_body,
        grid=(B, nsb),
        in_specs=[
            pl.BlockSpec((bm, D), lambda bb, i: (i, bb)),
            pl.BlockSpec((1, D), lambda bb, i: (0, 0)),
            pl.BlockSpec((1, D), lambda bb, i: (0, 0)),
        ],
        out_specs=pl.BlockSpec((bm, D), lambda bb, i: (bb * nsb + i, 0)),
        out_shape=jax.ShapeDtypeStruct((NT, D), jnp.bfloat16),
    )(x2d, w, b)


def _qkv_body(a_ref, w_ref, b_ref, o_ref):
    a = a_ref[...]
    w = w_ref[...]
    acc = lax.dot_general(a, w, (((1,), (1,)), ((), ())),
                          preferred_element_type=jnp.float32)
    acc = acc + b_ref[...]
    o_ref[...] = acc.astype(jnp.bfloat16)


def _qkv(xn, in_w, in_b2, bm=512, bn=1024):
    return pl.pallas_call(
        _qkv_body,
        grid=(3 * D // bn, NT // bm),
        in_specs=[
            pl.BlockSpec((bm, D), lambda cb, rb: (rb, 0)),
            pl.BlockSpec((bn, D), lambda cb, rb: (cb, 0)),
            pl.BlockSpec((1, bn), lambda cb, rb: (0, cb)),
        ],
        out_specs=pl.BlockSpec((bm, bn), lambda cb, rb: (rb, cb)),
        out_shape=jax.ShapeDtypeStruct((NT, 3 * D), jnp.bfloat16),
    )(xn, in_w, in_b2)


def _attn_body(q_ref, k_ref, v_ref, o_ref):
    qi = pl.program_id(2)
    q = q_ref[0, 0, 0, :, :]

    def block(j, carry, masked):
        o, m, l = carry
        kb = k_ref[0, 0, j, :, :]
        vb = v_ref[0, 0, j, :, :]
        s = lax.dot_general(q, kb, (((1,), (1,)), ((), ())),
                            preferred_element_type=jnp.float32) * SCALE
        if masked:
            row = lax.broadcasted_iota(jnp.int32, (BQ, BQ), 0)
            col = lax.broadcasted_iota(jnp.int32, (BQ, BQ), 1)
            s = jnp.where(col <= row, s, NEG)
        m_new = jnp.maximum(m, jnp.max(s, axis=-1, keepdims=True))
        alpha = jnp.exp(m - m_new)
        p = jnp.exp(s - m_new)
        l_new = l * alpha + jnp.sum(p, axis=-1, keepdims=True)
        o_new = o * alpha + lax.dot_general(
            p.astype(jnp.bfloat16), vb, (((1,), (0,)), ((), ())),
            preferred_element_type=jnp.float32)
        return o_new, m_new, l_new

    o0 = jnp.zeros((BQ, DH), jnp.float32)
    m0 = jnp.full((BQ, 1), NEG, jnp.float32)
    l0 = jnp.zeros((BQ, 1), jnp.float32)
    carry = lax.fori_loop(0, qi, lambda j, c: block(j, c, False), (o0, m0, l0))
    o, m, l = block(qi, carry, True)
    o_ref[0, 0, 0, :, :] = (o * (1.0 / l)).astype(jnp.bfloat16)


def _attn(qkvh):
    return pl.pallas_call(
        _attn_body,
        grid=(B, NH, NKB),
        in_specs=[
            pl.BlockSpec((1, 1, 1, BQ, DH), lambda b, h, qi: (b, h, qi, 0, 0)),
            pl.BlockSpec((1, 1, NKB, BQ, DH), lambda b, h, qi: (b, NH + h, 0, 0, 0)),
            pl.BlockSpec((1, 1, NKB, BQ, DH), lambda b, h, qi: (b, 2 * NH + h, 0, 0, 0)),
        ],
        out_specs=pl.BlockSpec((1, 1, 1, BQ, DH), lambda b, h, qi: (b, h, qi, 0, 0)),
        out_shape=jax.ShapeDtypeStruct((B, NH, NKB, BQ, DH), jnp.bfloat16),
    )(qkvh, qkvh, qkvh)


def _final_body(o_ref, w_ref, b_ref, x_ref, lw_ref, lb_ref, g_ref,
                out_ref, logit_ref):
    a = o_ref[...]
    w = w_ref[...]
    h = lax.dot_general(a, w, (((1,), (1,)), ((), ())),
                        preferred_element_type=jnp.float32)
    h = h + b_ref[...] + x_ref[...]
    m = jnp.mean(h, axis=-1, keepdims=True)
    v = jnp.mean((h - m) ** 2, axis=-1, keepdims=True)
    hs = (h - m) * lax.rsqrt(v + EPS) * lw_ref[...] + lb_ref[...]
    gp = g_ref[...]
    logits = lax.dot_general(hs, gp, (((1,), (1,)), ((), ())),
                             preferred_element_type=jnp.float32)
    logit_ref[...] = logits
    l0 = logits[:, 0:1]
    l1 = logits[:, 1:2]
    l2 = logits[:, 2:3]
    top = jnp.maximum(jnp.maximum(l0, l1), l2)
    bot = jnp.minimum(jnp.minimum(l0, l1), l2)
    mid = l0 + l1 + l2 - top - bot
    gate = 1.0 / (1.0 + jnp.exp(mid - top))
    out_ref[...] = h * gate


def _final(o2, out_w, out_b2, x2d, ln2_w2, ln2_b2, gate_pad, bm=512):
    nsb = S // bm
    return pl.pallas_call(
        _final_body,
        grid=(B, nsb),
        in_specs=[
            pl.BlockSpec((bm, D), lambda bb, i: (bb * nsb + i, 0)),
            pl.BlockSpec((D, D), lambda bb, i: (0, 0)),
            pl.BlockSpec((1, D), lambda bb, i: (0, 0)),
            pl.BlockSpec((bm, D), lambda bb, i: (i, bb)),
            pl.BlockSpec((1, D), lambda bb, i: (0, 0)),
            pl.BlockSpec((1, D), lambda bb, i: (0, 0)),
            pl.BlockSpec((128, D), lambda bb, i: (0, 0)),
        ],
        out_specs=[
            pl.BlockSpec((bm, D), lambda bb, i: (i, bb)),
            pl.BlockSpec((bm, 128), lambda bb, i: (i, bb)),
        ],
        out_shape=[
            jax.ShapeDtypeStruct((S, B * D), jnp.float32),
            jax.ShapeDtypeStruct((S, B * 128), jnp.float32),
        ],
    )(o2, out_w, out_b2, x2d, ln2_w2, ln2_b2, gate_pad)


def kernel(x, ln1_w, ln1_b, in_w, in_b, out_w, out_b, ln2_w, ln2_b, gate_w):
    x2d = x.astype(jnp.float32).reshape(S, B * D)
    xn = _ln1(x2d, ln1_w.reshape(1, D), ln1_b.reshape(1, D))
    qkv = _qkv(xn, in_w.astype(jnp.bfloat16), in_b.reshape(1, 3 * D))
    qkvh = qkv.reshape(B, S, 3 * NH, DH).transpose(0, 2, 1, 3)
    qkvh = qkvh.reshape(B, 3 * NH, NKB, BQ, DH)
    o = _attn(qkvh)
    o2 = o.reshape(B, NH, S, DH).transpose(0, 2, 1, 3).reshape(NT, D)
    gate_pad = jnp.zeros((128, D), jnp.float32).at[:NE].set(gate_w)
    out2d, logits2d = _final(o2, out_w.astype(jnp.bfloat16), out_b.reshape(1, D),
                             x2d, ln2_w.reshape(1, D), ln2_b.reshape(1, D),
                             gate_pad)
    out = out2d.reshape(S, B, D)
    logits = logits2d.reshape(S, B, 128)[:, :, :NE].reshape(NT, NE)
    return out, logits


# speedup vs baseline: 1.7904x; 1.0534x over previous
import functools
import jax
import jax.numpy as jnp
from jax import lax
from jax.experimental import pallas as pl
from jax.experimental.pallas import tpu as pltpu

D = 2048
NH = 16
DH = D // NH
S = 2048
B = 2
NE = 3
NT = S * B
EPS = 1e-5
SCALE = 1.0 / (DH ** 0.5)
NEG = -1e30
BQ = 512
NKB = S // BQ


def _ln1_body(x_ref, w_ref, b_ref, o_ref):
    x = x_ref[...]
    m = jnp.mean(x, axis=-1, keepdims=True)
    v = jnp.mean((x - m) ** 2, axis=-1, keepdims=True)
    y = (x - m) * lax.rsqrt(v + EPS) * w_ref[...] + b_ref[...]
    o_ref[...] = y.astype(jnp.bfloat16)


def _ln1(x2d, w, b, bm=512):
    nsb = S // bm
    return pl.pallas_call(
        _ln1_body,
        grid=(B, nsb),
        in_specs=[
            pl.BlockSpec((bm, D), lambda bb, i: (i, bb)),
            pl.BlockSpec((1, D), lambda bb, i: (0, 0)),
            pl.BlockSpec((1, D), lambda bb, i: (0, 0)),
        ],
        out_specs=pl.BlockSpec((bm, D), lambda bb, i: (bb * nsb + i, 0)),
        out_shape=jax.ShapeDtypeStruct((NT, D), jnp.bfloat16),
    )(x2d, w, b)


def _qkv_body(a_ref, w_ref, b_ref, o_ref):
    a = a_ref[...]
    w = w_ref[...]
    acc = lax.dot_general(a, w, (((1,), (1,)), ((), ())),
                          preferred_element_type=jnp.float32)
    acc = acc + b_ref[...]
    o_ref[...] = acc.astype(jnp.bfloat16)


def _qkv(xn, in_w, in_b2, bm=512, bn=1024):
    return pl.pallas_call(
        _qkv_body,
        grid=(3 * D // bn, NT // bm),
        in_specs=[
            pl.BlockSpec((bm, D), lambda cb, rb: (rb, 0)),
            pl.BlockSpec((bn, D), lambda cb, rb: (cb, 0)),
            pl.BlockSpec((1, bn), lambda cb, rb: (0, cb)),
        ],
        out_specs=pl.BlockSpec((bm, bn), lambda cb, rb: (rb, cb)),
        out_shape=jax.ShapeDtypeStruct((NT, 3 * D), jnp.bfloat16),
    )(xn, in_w, in_b2)


HPP = 2


def _attn_body(q_ref, k_ref, v_ref, o_ref):
    qi = pl.program_id(2)

    def block(hh, j, carry, masked):
        o, m, l = carry
        q = q_ref[0, hh, 0, :, :]
        kb = k_ref[0, hh, j, :, :]
        vb = v_ref[0, hh, j, :, :]
        s = lax.dot_general(q, kb, (((1,), (1,)), ((), ())),
                            preferred_element_type=jnp.float32) * SCALE
        if masked:
            row = lax.broadcasted_iota(jnp.int32, (BQ, BQ), 0)
            col = lax.broadcasted_iota(jnp.int32, (BQ, BQ), 1)
            s = jnp.where(col <= row, s, NEG)
        m_new = jnp.maximum(m, jnp.max(s, axis=-1, keepdims=True))
        alpha = jnp.exp(m - m_new)
        p = jnp.exp(s - m_new)
        l_new = l * alpha + jnp.sum(p, axis=-1, keepdims=True)
        o_new = o * alpha + lax.dot_general(
            p.astype(jnp.bfloat16), vb, (((1,), (0,)), ((), ())),
            preferred_element_type=jnp.float32)
        return o_new, m_new, l_new

    def init():
        return (jnp.zeros((BQ, DH), jnp.float32),
                jnp.full((BQ, 1), NEG, jnp.float32),
                jnp.zeros((BQ, 1), jnp.float32))

    def step(j, carries):
        return tuple(block(hh, j, c, False) for hh, c in enumerate(carries))

    carries = lax.fori_loop(0, qi, step, tuple(init() for _ in range(HPP)))
    for hh in range(HPP):
        o, m, l = block(hh, qi, carries[hh], True)
        o_ref[0, hh, 0, :, :] = (o * (1.0 / l)).astype(jnp.bfloat16)


def _attn(qkvh):
    return pl.pallas_call(
        _attn_body,
        grid=(B, NH // HPP, NKB),
        in_specs=[
            pl.BlockSpec((1, HPP, 1, BQ, DH),
                         lambda b, h, qi: (b, h, qi, 0, 0)),
            pl.BlockSpec((1, HPP, NKB, BQ, DH),
                         lambda b, h, qi: (b, NH // HPP + h, 0, 0, 0)),
            pl.BlockSpec((1, HPP, NKB, BQ, DH),
                         lambda b, h, qi: (b, 2 * NH // HPP + h, 0, 0, 0)),
        ],
        out_specs=pl.BlockSpec((1, HPP, 1, BQ, DH),
                               lambda b, h, qi: (b, h, qi, 0, 0)),
        out_shape=jax.ShapeDtypeStruct((B, NH, NKB, BQ, DH), jnp.bfloat16),
    )(qkvh, qkvh, qkvh)


def _final_body(o_ref, w_ref, b_ref, x_ref, lw_ref, lb_ref, g_ref,
                out_ref, logit_ref):
    a = o_ref[...]
    w = w_ref[...]
    h = lax.dot_general(a, w, (((1,), (1,)), ((), ())),
                        preferred_element_type=jnp.float32)
    h = h + b_ref[...] + x_ref[...]
    m = jnp.mean(h, axis=-1, keepdims=True)
    v = jnp.mean((h - m) ** 2, axis=-1, keepdims=True)
    hs = (h - m) * lax.rsqrt(v + EPS) * lw_ref[...] + lb_ref[...]
    gp = g_ref[...]
    logits = lax.dot_general(hs, gp, (((1,), (1,)), ((), ())),
                             preferred_element_type=jnp.float32)
    logit_ref[...] = logits
    l0 = logits[:, 0:1]
    l1 = logits[:, 1:2]
    l2 = logits[:, 2:3]
    top = jnp.maximum(jnp.maximum(l0, l1), l2)
    bot = jnp.minimum(jnp.minimum(l0, l1), l2)
    mid = l0 + l1 + l2 - top - bot
    gate = 1.0 / (1.0 + jnp.exp(mid - top))
    out_ref[...] = h * gate


def _final(o2, out_w, out_b2, x2d, ln2_w2, ln2_b2, gate_pad, bm=512):
    nsb = S // bm
    return pl.pallas_call(
        _final_body,
        grid=(B, nsb),
        in_specs=[
            pl.BlockSpec((bm, D), lambda bb, i: (bb * nsb + i, 0)),
            pl.BlockSpec((D, D), lambda bb, i: (0, 0)),
            pl.BlockSpec((1, D), lambda bb, i: (0, 0)),
            pl.BlockSpec((bm, D), lambda bb, i: (i, bb)),
            pl.BlockSpec((1, D), lambda bb, i: (0, 0)),
            pl.BlockSpec((1, D), lambda bb, i: (0, 0)),
            pl.BlockSpec((128, D), lambda bb, i: (0, 0)),
        ],
        out_specs=[
            pl.BlockSpec((bm, D), lambda bb, i: (i, bb)),
            pl.BlockSpec((bm, 128), lambda bb, i: (i, bb)),
        ],
        out_shape=[
            jax.ShapeDtypeStruct((S, B * D), jnp.float32),
            jax.ShapeDtypeStruct((S, B * 128), jnp.float32),
        ],
    )(o2, out_w, out_b2, x2d, ln2_w2, ln2_b2, gate_pad)


def kernel(x, ln1_w, ln1_b, in_w, in_b, out_w, out_b, ln2_w, ln2_b, gate_w):
    x2d = x.astype(jnp.float32).reshape(S, B * D)
    xn = _ln1(x2d, ln1_w.reshape(1, D), ln1_b.reshape(1, D))
    qkv = _qkv(xn, in_w.astype(jnp.bfloat16), in_b.reshape(1, 3 * D))
    qkvh = qkv.reshape(B, S, 3 * NH, DH).transpose(0, 2, 1, 3)
    qkvh = qkvh.reshape(B, 3 * NH, NKB, BQ, DH)
    o = _attn(qkvh)
    o2 = o.reshape(B, NH, S, DH).transpose(0, 2, 1, 3).reshape(NT, D)
    gate_pad = jnp.zeros((128, D), jnp.float32).at[:NE].set(gate_w)
    out2d, logits2d = _final(o2, out_w.astype(jnp.bfloat16), out_b.reshape(1, D),
                             x2d, ln2_w.reshape(1, D), ln2_b.reshape(1, D),
                             gate_pad)
    out = out2d.reshape(S, B, D)
    logits = logits2d.reshape(S, B, 128)[:, :, :NE].reshape(NT, NE)
    return out, logits
